# baseline (speedup 1.0000x reference)
import jax
import jax.numpy as jnp
from jax import lax
from jax.experimental import pallas as pl
from jax.experimental.pallas import tpu as pltpu
from jax.experimental.pallas import tpu_sc as plsc

H = 64
NV = H // 16
SEQ = 200
CHUNK = 100
NC = 2
NS = 16
NW = NC * NS
NBUF = 4


def _lane_sum(v):
  lanes = lax.iota(jnp.int32, 16)
  for step in (8, 4, 2, 1):
    perm = lanes ^ step
    v = v + v.at[perm].get(mode="promise_in_bounds")
  return v


def _rsqrt(x):
  i = lax.bitcast_convert_type(x, jnp.int32)
  i = jnp.int32(0x5F3759DF) - (i >> 1)
  y = lax.bitcast_convert_type(i, jnp.float32)
  for _ in range(2):
    y = y * (1.5 - 0.5 * x * y * y)
  return y


def _body(ids_hbm, table_hbm, pos_hbm, gam_hbm, bet_hbm, out_hbm,
          idx_all, rows, pos_v, gam_v, bet_v, gsems, osems):
  cpw = ids_hbm.shape[0] // NW
  wid = lax.axis_index("s") * NC + lax.axis_index("c")
  base = wid * cpw

  pltpu.sync_copy(ids_hbm.at[pl.ds(base, cpw)], idx_all)
  pltpu.sync_copy(pos_hbm, pos_v)
  pltpu.sync_copy(gam_hbm, gam_v)
  pltpu.sync_copy(bet_hbm, bet_v)
  g = [gam_v[pl.ds(16 * j, 16)] for j in range(NV)]
  b = [bet_v[pl.ds(16 * j, 16)] for j in range(NV)]

  pltpu.async_copy(table_hbm.at[idx_all.at[0]], rows[0], gsems[0])
  pltpu.async_copy(table_hbm.at[idx_all.at[1]], rows[1], gsems[1])

  def compute_chunk(rv, off):
    @plsc.parallel_loop(0, CHUNK, unroll=10)
    def row_body(i):
      x = [rv[i, pl.ds(16 * j, 16)] + pos_v[off + i, pl.ds(16 * j, 16)]
           for j in range(NV)]
      s = (x[0] + x[1]) + (x[2] + x[3])
      q = (x[0] * x[0] + x[1] * x[1]) + (x[2] * x[2] + x[3] * x[3])
      mean = _lane_sum(s) * (1.0 / H)
      var = _lane_sum(q) * (1.0 / H) - mean * mean
      inv = _rsqrt(var + 1e-12)
      for j in range(NV):
        ig = inv * g[j]
        rv[i, pl.ds(16 * j, 16)] = x[j] * ig + (b[j] - mean * ig)

  def ring_body(it, carry):
    for bb in range(NBUF):
      c = it * NBUF + bb
      b2 = (bb + 2) % NBUF

      @pl.when((c >= 2) & (c + 2 < cpw))
      def _():
        pltpu.make_async_copy(
            rows[b2], out_hbm.at[base + c - 2], osems[b2]).wait()

      @pl.when(c + 2 < cpw)
      def _():
        pltpu.async_copy(
            table_hbm.at[idx_all.at[c + 2]], rows[b2], gsems[b2])

      pltpu.make_async_copy(
          table_hbm.at[idx_all.at[c]], rows[bb], gsems[bb]).wait()
      compute_chunk(rows[bb], (bb % 2) * CHUNK)
      pltpu.async_copy(rows[bb], out_hbm.at[base + c], osems[bb])
    return carry

  lax.fori_loop(0, cpw // NBUF, ring_body, 0)

  for bb in range(NBUF):
    pltpu.make_async_copy(
        rows[bb], out_hbm.at[base + cpw - NBUF + bb], osems[bb]).wait()


def kernel(input_ids, item_table, pos_table, ln_gamma, ln_beta):
  batch, seq = input_ids.shape
  rows = batch * seq
  nchunks = rows // CHUNK
  ids2 = input_ids.astype(jnp.int32).reshape(nchunks, CHUNK)

  def body(ids_hbm, table_hbm, pos_hbm, gam_hbm, bet_hbm, out_hbm,
           idx_all, r0, r1, r2, r3, pos_v, gam_v, bet_v,
           g0, g1, g2, g3, o0, o1, o2, o3):
    _body(ids_hbm, table_hbm, pos_hbm, gam_hbm, bet_hbm, out_hbm,
          idx_all, [r0, r1, r2, r3], pos_v, gam_v, bet_v,
          [g0, g1, g2, g3], [o0, o1, o2, o3])

  mesh = plsc.VectorSubcoreMesh(core_axis_name="c", subcore_axis_name="s")
  run = pl.kernel(
      body,
      mesh=mesh,
      compiler_params=pltpu.CompilerParams(use_tc_tiling_on_sc=False),
      out_type=jax.ShapeDtypeStruct((nchunks, CHUNK, H), jnp.float32),
      scratch_types=(
          [pltpu.VMEM((nchunks // NW, CHUNK), jnp.int32)]
          + [pltpu.VMEM((CHUNK, H), jnp.float32) for _ in range(NBUF)]
          + [pltpu.VMEM((SEQ, H), jnp.float32),
             pltpu.VMEM((H,), jnp.float32),
             pltpu.VMEM((H,), jnp.float32)]
          + [pltpu.SemaphoreType.DMA for _ in range(2 * NBUF)]
      ),
  )
  out = run(ids2, item_table, pos_table, ln_gamma, ln_beta)
  return out.reshape(batch, seq, H)


# speedup vs baseline: 1.1177x; 1.1177x over previous
import jax
import jax.numpy as jnp
from jax import lax
from jax.experimental import pallas as pl
from jax.experimental.pallas import tpu as pltpu
from jax.experimental.pallas import tpu_sc as plsc

H = 64
NV = H // 16
SEQ = 200
CHUNK = 100
NC = 2
NS = 16
NW = NC * NS
NBUF = 4


def _lane_sum(v):
  lanes = lax.iota(jnp.int32, 16)
  for step in (8, 4, 2, 1):
    perm = lanes ^ step
    v = v + v.at[perm].get(mode="promise_in_bounds")
  return v


def _rsqrt(x):
  i = lax.bitcast_convert_type(x, jnp.int32)
  i = jnp.int32(0x5F3759DF) - (i >> 1)
  y = lax.bitcast_convert_type(i, jnp.float32)
  for _ in range(2):
    y = y * (1.5 - 0.5 * x * y * y)
  return y


def _body(ids_hbm, table_hbm, pos_hbm, gam_hbm, bet_hbm, out_hbm,
          idx_all, rows, pos_v, gam_v, bet_v, gsems, osems):
  cpw = ids_hbm.shape[0] // NW
  wid = lax.axis_index("s") * NC + lax.axis_index("c")
  base = wid * cpw

  pltpu.sync_copy(ids_hbm.at[pl.ds(base, cpw)], idx_all)
  pltpu.sync_copy(pos_hbm, pos_v)
  pltpu.sync_copy(gam_hbm, gam_v)
  pltpu.sync_copy(bet_hbm, bet_v)
  g = [gam_v[pl.ds(16 * j, 16)] for j in range(NV)]
  b = [bet_v[pl.ds(16 * j, 16)] for j in range(NV)]

  pltpu.async_copy(table_hbm.at[idx_all.at[0]], rows[0], gsems[0])
  pltpu.async_copy(table_hbm.at[idx_all.at[1]], rows[1], gsems[1])

  def compute_chunk(rv, off):
    @plsc.parallel_loop(0, CHUNK, unroll=5)
    def row_body(i):
      x = [rv[i, pl.ds(16 * j, 16)] + pos_v[off + i, pl.ds(16 * j, 16)]
           for j in range(NV)]
      s = (x[0] + x[1]) + (x[2] + x[3])
      q = (x[0] * x[0] + x[1] * x[1]) + (x[2] * x[2] + x[3] * x[3])
      mean = _lane_sum(s) * (1.0 / H)
      var = _lane_sum(q) * (1.0 / H) - mean * mean
      inv = _rsqrt(var + 1e-12)
      for j in range(NV):
        ig = inv * g[j]
        rv[i, pl.ds(16 * j, 16)] = x[j] * ig + (b[j] - mean * ig)

  def ring_body(it, carry):
    for bb in range(NBUF):
      c = it * NBUF + bb
      b2 = (bb + 2) % NBUF

      @pl.when((c >= 2) & (c + 2 < cpw))
      def _():
        pltpu.make_async_copy(
            rows[b2], out_hbm.at[base + c - 2], osems[b2]).wait()

      @pl.when(c + 2 < cpw)
      def _():
        pltpu.async_copy(
            table_hbm.at[idx_all.at[c + 2]], rows[b2], gsems[b2])

      pltpu.make_async_copy(
          table_hbm.at[idx_all.at[c]], rows[bb], gsems[bb]).wait()
      compute_chunk(rows[bb], (bb % 2) * CHUNK)
      pltpu.async_copy(rows[bb], out_hbm.at[base + c], osems[bb])
    return carry

  lax.fori_loop(0, cpw // NBUF, ring_body, 0)

  for bb in range(NBUF):
    pltpu.make_async_copy(
        rows[bb], out_hbm.at[base + cpw - NBUF + bb], osems[bb]).wait()


def kernel(input_ids, item_table, pos_table, ln_gamma, ln_beta):
  batch, seq = input_ids.shape
  rows = batch * seq
  nchunks = rows // CHUNK
  ids2 = input_ids.astype(jnp.int32).reshape(nchunks, CHUNK)

  def body(ids_hbm, table_hbm, pos_hbm, gam_hbm, bet_hbm, out_hbm,
           idx_all, r0, r1, r2, r3, pos_v, gam_v, bet_v,
           g0, g1, g2, g3, o0, o1, o2, o3):
    _body(ids_hbm, table_hbm, pos_hbm, gam_hbm, bet_hbm, out_hbm,
          idx_all, [r0, r1, r2, r3], pos_v, gam_v, bet_v,
          [g0, g1, g2, g3], [o0, o1, o2, o3])

  mesh = plsc.VectorSubcoreMesh(core_axis_name="c", subcore_axis_name="s")
  run = pl.kernel(
      body,
      mesh=mesh,
      compiler_params=pltpu.CompilerParams(use_tc_tiling_on_sc=False),
      out_type=jax.ShapeDtypeStruct((nchunks, CHUNK, H), jnp.float32),
      scratch_types=(
          [pltpu.VMEM((nchunks // NW, CHUNK), jnp.int32)]
          + [pltpu.VMEM((CHUNK, H), jnp.float32) for _ in range(NBUF)]
          + [pltpu.VMEM((SEQ, H), jnp.float32),
             pltpu.VMEM((H,), jnp.float32),
             pltpu.VMEM((H,), jnp.float32)]
          + [pltpu.SemaphoreType.DMA for _ in range(2 * NBUF)]
      ),
  )
  out = run(ids2, item_table, pos_table, ln_gamma, ln_beta)
  return out.reshape(batch, seq, H)


# speedup vs baseline: 1.2680x; 1.1345x over previous
import jax
import jax.numpy as jnp
from jax import lax
from jax.experimental import pallas as pl
from jax.experimental.pallas import tpu as pltpu
from jax.experimental.pallas import tpu_sc as plsc

H = 64
NV = H // 16
SEQ = 200
CHUNK = 100
NC = 2
NS = 16
NW = NC * NS
NBUF = 4


def _lane_sum(v):
  ps = plsc.cumsum(v)
  last = lax.iota(jnp.int32, 16) | 15
  return ps.at[last].get(mode="promise_in_bounds")


def _rsqrt(x):
  i = lax.bitcast_convert_type(x, jnp.int32)
  i = jnp.int32(0x5F3759DF) - (i >> 1)
  y = lax.bitcast_convert_type(i, jnp.float32)
  for _ in range(2):
    y = y * (1.5 - 0.5 * x * y * y)
  return y


def _body(ids_hbm, table_hbm, pos_hbm, gam_hbm, bet_hbm, out_hbm,
          idx_all, rows, pos_v, gam_v, bet_v, gsems, osems):
  cpw = ids_hbm.shape[0] // NW
  wid = lax.axis_index("s") * NC + lax.axis_index("c")
  base = wid * cpw

  pltpu.sync_copy(ids_hbm.at[pl.ds(base, cpw)], idx_all)
  pltpu.sync_copy(pos_hbm, pos_v)
  pltpu.sync_copy(gam_hbm, gam_v)
  pltpu.sync_copy(bet_hbm, bet_v)
  g = [gam_v[pl.ds(16 * j, 16)] for j in range(NV)]
  b = [bet_v[pl.ds(16 * j, 16)] for j in range(NV)]

  pltpu.async_copy(table_hbm.at[idx_all.at[0]], rows[0], gsems[0])
  pltpu.async_copy(table_hbm.at[idx_all.at[1]], rows[1], gsems[1])

  def compute_chunk(rv, off):
    @plsc.parallel_loop(0, CHUNK, unroll=5)
    def row_body(i):
      x = [rv[i, pl.ds(16 * j, 16)] + pos_v[off + i, pl.ds(16 * j, 16)]
           for j in range(NV)]
      s = (x[0] + x[1]) + (x[2] + x[3])
      q = (x[0] * x[0] + x[1] * x[1]) + (x[2] * x[2] + x[3] * x[3])
      mean = _lane_sum(s) * (1.0 / H)
      var = _lane_sum(q) * (1.0 / H) - mean * mean
      inv = _rsqrt(var + 1e-12)
      for j in range(NV):
        rv[i, pl.ds(16 * j, 16)] = (x[j] - mean) * inv * g[j] + b[j]

  def ring_body(it, carry):
    for bb in range(NBUF):
      c = it * NBUF + bb
      b2 = (bb + 2) % NBUF

      @pl.when((c >= 2) & (c + 2 < cpw))
      def _():
        pltpu.make_async_copy(
            rows[b2], out_hbm.at[base + c - 2], osems[b2]).wait()

      @pl.when(c + 2 < cpw)
      def _():
        pltpu.async_copy(
            table_hbm.at[idx_all.at[c + 2]], rows[b2], gsems[b2])

      pltpu.make_async_copy(
          table_hbm.at[idx_all.at[c]], rows[bb], gsems[bb]).wait()
      compute_chunk(rows[bb], (bb % 2) * CHUNK)
      pltpu.async_copy(rows[bb], out_hbm.at[base + c], osems[bb])
    return carry

  lax.fori_loop(0, cpw // NBUF, ring_body, 0)

  for bb in range(NBUF):
    pltpu.make_async_copy(
        rows[bb], out_hbm.at[base + cpw - NBUF + bb], osems[bb]).wait()


def kernel(input_ids, item_table, pos_table, ln_gamma, ln_beta):
  batch, seq = input_ids.shape
  rows = batch * seq
  nchunks = rows // CHUNK
  ids2 = input_ids.astype(jnp.int32).reshape(nchunks, CHUNK)

  def body(ids_hbm, table_hbm, pos_hbm, gam_hbm, bet_hbm, out_hbm,
           idx_all, r0, r1, r2, r3, pos_v, gam_v, bet_v,
           g0, g1, g2, g3, o0, o1, o2, o3):
    _body(ids_hbm, table_hbm, pos_hbm, gam_hbm, bet_hbm, out_hbm,
          idx_all, [r0, r1, r2, r3], pos_v, gam_v, bet_v,
          [g0, g1, g2, g3], [o0, o1, o2, o3])

  mesh = plsc.VectorSubcoreMesh(core_axis_name="c", subcore_axis_name="s")
  run = pl.kernel(
      body,
      mesh=mesh,
      compiler_params=pltpu.CompilerParams(
          use_tc_tiling_on_sc=False, needs_layout_passes=False),
      out_type=jax.ShapeDtypeStruct((nchunks, CHUNK, H), jnp.float32),
      scratch_types=(
          [pltpu.VMEM((nchunks // NW, CHUNK), jnp.int32)]
          + [pltpu.VMEM((CHUNK, H), jnp.float32) for _ in range(NBUF)]
          + [pltpu.VMEM((SEQ, H), jnp.float32),
             pltpu.VMEM((H,), jnp.float32),
             pltpu.VMEM((H,), jnp.float32)]
          + [pltpu.SemaphoreType.DMA for _ in range(2 * NBUF)]
      ),
  )
  out = run(ids2, item_table, pos_table, ln_gamma, ln_beta)
  return out.reshape(batch, seq, H)


# speedup vs baseline: 1.2685x; 1.0004x over previous
import jax
import jax.numpy as jnp
from jax import lax
from jax.experimental import pallas as pl
from jax.experimental.pallas import tpu as pltpu
from jax.experimental.pallas import tpu_sc as plsc

H = 64
NV = H // 16
SEQ = 200
CHUNK = 100
NC = 2
NS = 16
NW = NC * NS
NBUF = 4


def _lane_sum(v):
  ps = plsc.cumsum(v)
  last = lax.iota(jnp.int32, 16) | 15
  return ps.at[last].get(mode="promise_in_bounds")


def _rsqrt(x):
  i = lax.bitcast_convert_type(x, jnp.int32)
  i = jnp.int32(0x5F3759DF) - (i >> 1)
  y = lax.bitcast_convert_type(i, jnp.float32)
  for _ in range(2):
    y = y * (1.5 - 0.5 * x * y * y)
  return y


def _body(ids_hbm, table_hbm, pos_hbm, gam_hbm, bet_hbm, out_hbm,
          idx_all, rows, pos_v, gam_v, bet_v, gsems, osems):
  cpw = ids_hbm.shape[0] // NW
  wid = lax.axis_index("s") * NC + lax.axis_index("c")
  base = wid * cpw

  pltpu.sync_copy(ids_hbm.at[pl.ds(base, cpw)], idx_all)
  pltpu.sync_copy(pos_hbm, pos_v)
  pltpu.sync_copy(gam_hbm, gam_v)
  pltpu.sync_copy(bet_hbm, bet_v)
  g = [gam_v[pl.ds(16 * j, 16)] for j in range(NV)]
  b = [bet_v[pl.ds(16 * j, 16)] for j in range(NV)]

  pltpu.async_copy(table_hbm.at[idx_all.at[0]], rows[0], gsems[0])
  pltpu.async_copy(table_hbm.at[idx_all.at[1]], rows[1], gsems[1])

  def compute_chunk(rv, off):
    @plsc.parallel_loop(0, CHUNK, unroll=5)
    def row_body(i):
      x = [rv[i, pl.ds(16 * j, 16)] + pos_v[off + i, pl.ds(16 * j, 16)]
           for j in range(NV)]
      s = (x[0] + x[1]) + (x[2] + x[3])
      q = (x[0] * x[0] + x[1] * x[1]) + (x[2] * x[2] + x[3] * x[3])
      mean = _lane_sum(s) * (1.0 / H)
      var = _lane_sum(q) * (1.0 / H) - mean * mean
      inv = _rsqrt(var + 1e-12)
      for j in range(NV):
        rv[i, pl.ds(16 * j, 16)] = (x[j] - mean) * inv * g[j] + b[j]

  def oref(r):
    return out_hbm.at[r // 2, pl.ds((r % 2) * CHUNK, CHUNK)]

  def ring_body(it, carry):
    for bb in range(NBUF):
      c = it * NBUF + bb
      b2 = (bb + 2) % NBUF

      @pl.when((c >= 2) & (c + 2 < cpw))
      def _():
        pltpu.make_async_copy(rows[b2], oref(base + c - 2), osems[b2]).wait()

      @pl.when(c + 2 < cpw)
      def _():
        pltpu.async_copy(
            table_hbm.at[idx_all.at[c + 2]], rows[b2], gsems[b2])

      pltpu.make_async_copy(
          table_hbm.at[idx_all.at[c]], rows[bb], gsems[bb]).wait()
      compute_chunk(rows[bb], (bb % 2) * CHUNK)
      pltpu.async_copy(rows[bb], oref(base + c), osems[bb])
    return carry

  lax.fori_loop(0, cpw // NBUF, ring_body, 0)

  for bb in range(NBUF):
    pltpu.make_async_copy(
        rows[bb], oref(base + cpw - NBUF + bb), osems[bb]).wait()


def kernel(input_ids, item_table, pos_table, ln_gamma, ln_beta):
  batch, seq = input_ids.shape
  rows = batch * seq
  nchunks = rows // CHUNK
  ids2 = input_ids.astype(jnp.int32).reshape(nchunks, CHUNK)

  def body(ids_hbm, table_hbm, pos_hbm, gam_hbm, bet_hbm, out_hbm,
           idx_all, r0, r1, r2, r3, pos_v, gam_v, bet_v,
           g0, g1, g2, g3, o0, o1, o2, o3):
    _body(ids_hbm, table_hbm, pos_hbm, gam_hbm, bet_hbm, out_hbm,
          idx_all, [r0, r1, r2, r3], pos_v, gam_v, bet_v,
          [g0, g1, g2, g3], [o0, o1, o2, o3])

  mesh = plsc.VectorSubcoreMesh(core_axis_name="c", subcore_axis_name="s")
  run = pl.kernel(
      body,
      mesh=mesh,
      compiler_params=pltpu.CompilerParams(
          use_tc_tiling_on_sc=False, needs_layout_passes=False),
      out_type=jax.ShapeDtypeStruct((batch, seq, H), jnp.float32),
      scratch_types=(
          [pltpu.VMEM((nchunks // NW, CHUNK), jnp.int32)]
          + [pltpu.VMEM((CHUNK, H), jnp.float32) for _ in range(NBUF)]
          + [pltpu.VMEM((SEQ, H), jnp.float32),
             pltpu.VMEM((H,), jnp.float32),
             pltpu.VMEM((H,), jnp.float32)]
          + [pltpu.SemaphoreType.DMA for _ in range(2 * NBUF)]
      ),
  )
  return run(ids2, item_table, pos_table, ln_gamma, ln_beta)
